# baseline (speedup 1.0000x reference)
import jax
import jax.numpy as jnp
from jax import lax
from jax.experimental import pallas as pl
from jax.experimental.pallas import tpu as pltpu
from jax.experimental.pallas import tpu_sc as plsc

NUM_D = 383
NUM_M = 495
NUM_N = NUM_D + NUM_M
DIM = 128
HIDDEN = 512
N_PAIRS = 262144

ROW_PAD = 880
COL_PAD = 896
RB = ROW_PAD // 8
CB = COL_PAD // 128
N_TILES = RB * CB
TABLE_LEN = N_TILES * 1024

NUM_CORES = 2
NUM_SUBCORES = 16
NUM_WORKERS = NUM_CORES * NUM_SUBCORES
CHUNK = N_PAIRS // NUM_WORKERS
LANES = 16

N_SUB = 4
SUB = CHUNK // N_SUB


def _elu(x):
    return jnp.where(x > 0, x, jnp.exp(x) - 1.0)


def _tc_score_table(zd_ref, zm_ref, dsim_ref, msim_ref,
                    aw1_ref, ab1_ref, aw2_ref,
                    dfc_ref, db_ref, mfc_ref, mb_ref,
                    hw_ref, hb_ref, bil_ref, out_ref):
    bf16 = jnp.bfloat16
    aw1 = aw1_ref[...].astype(bf16)
    ab1 = ab1_ref[...]
    aw2 = aw2_ref[...][None, :]

    def attn(z_ref, n):
        betas = []
        for p in range(5):
            zp = z_ref[p]
            w = jnp.tanh(jnp.dot(zp.astype(bf16), aw1,
                                 preferred_element_type=jnp.float32) + ab1)
            s = jnp.sum(w * aw2, axis=1, keepdims=True)
            betas.append(jax.nn.sigmoid(jnp.sum(s) / n))
        h = betas[0] * z_ref[0]
        for p in range(1, 5):
            h = h + betas[p] * z_ref[p]
        return h

    h1 = attn(zd_ref, NUM_D)
    h2 = attn(zm_ref, NUM_M)

    h_d = _elu(jnp.dot(h1, dfc_ref[:DIM], preferred_element_type=jnp.float32)
               + jnp.dot(dsim_ref[...], dfc_ref[DIM:],
                         preferred_element_type=jnp.float32)
               + db_ref[...])
    h_m = _elu(jnp.dot(h2, mfc_ref[:DIM], preferred_element_type=jnp.float32)
               + jnp.dot(msim_ref[...], mfc_ref[DIM:],
                         preferred_element_type=jnp.float32)
               + mb_ref[...])

    pad2 = jnp.zeros((ROW_PAD - NUM_N, DIM), jnp.float32)
    h = jnp.concatenate([h_d, h_m, pad2], axis=0)
    h = _elu(jnp.dot(h, hw_ref[...], preferred_element_type=jnp.float32)
             + hb_ref[...])
    g = jnp.dot(h, bil_ref[...], preferred_element_type=jnp.float32)
    scores = lax.dot_general(g, h, (((1,), (1,)), ((), ())),
                             preferred_element_type=jnp.float32)
    scores = jax.nn.sigmoid(scores)
    scores = jnp.concatenate(
        [scores, jnp.zeros((ROW_PAD, COL_PAD - ROW_PAD), jnp.float32)],
        axis=1)
    for rb in range(RB):
        for cb in range(CB):
            out_ref[rb * CB + cb] = scores[8 * rb:8 * rb + 8,
                                           128 * cb:128 * cb + 128]


STAGE = TABLE_LEN // NUM_SUBCORES


def _sc_gather(sflat_hbm, d_hbm, m_hbm, out_hbm, d_v, m_v, idx_v, val_v, stab,
               ld_sem, tab_sem, g_sem, st_sem):
    sid = lax.axis_index("s")
    wid = sid * NUM_CORES + lax.axis_index("c")
    base = wid * CHUNK
    toff = pl.multiple_of(sid * STAGE, 8)
    tstage = pltpu.async_copy(sflat_hbm.at[pl.ds(toff, STAGE)],
                              stab.at[pl.ds(toff, STAGE)], tab_sem)
    ld_d = pltpu.async_copy(d_hbm.at[pl.ds(base, CHUNK)], d_v, ld_sem)
    ld_m = pltpu.async_copy(m_hbm.at[pl.ds(base, CHUNK)], m_v, ld_sem)
    ld_d.wait()
    ld_m.wait()

    vec_per_iter = 8
    n_iter = SUB // (LANES * vec_per_iter)

    gathers = []
    stores = []
    for k in range(N_SUB):
        kbase = k * SUB

        def body(i, carry, kbase=kbase):
            for j in range(vec_per_iter):
                off = pl.multiple_of(
                    kbase + i * (LANES * vec_per_iter) + j * LANES, LANES)
                r = d_v[pl.ds(off, LANES)]
                c = m_v[pl.ds(off, LANES)]
                tile = (r >> 3) * CB + (c >> 7)
                idx_v[pl.ds(off, LANES)] = ((tile << 10) + ((r & 7) << 7)
                                            + (c & 127))
            return carry

        lax.fori_loop(0, n_iter, body, 0)
        if k == 0:
            tstage.wait()
            plsc.subcore_barrier()
        gathers.append(pltpu.async_copy(
            stab.at[idx_v.at[pl.ds(kbase, SUB)]],
            val_v.at[pl.ds(kbase, SUB)], g_sem))
    for k in range(N_SUB):
        gathers[k].wait()
        stores.append(pltpu.async_copy(
            val_v.at[pl.ds(k * SUB, SUB)],
            out_hbm.at[pl.ds(base + k * SUB, SUB)], st_sem))
    for s in stores:
        s.wait()


def kernel(z_d, z_m, d_sim, m_sim, diseases, mirnas, att_w1, att_b1, att_w2,
           mfc_w, mfc_b, dfc_w, dfc_b, hfc_w, hfc_b, bil_w):
    f32 = jnp.float32

    zd_t = jnp.transpose(z_d, (1, 0, 2))
    zm_t = jnp.transpose(z_m, (1, 0, 2))
    aw2 = att_w2.reshape(HIDDEN)

    table = pl.pallas_call(
        _tc_score_table,
        out_shape=jax.ShapeDtypeStruct((N_TILES, 8, 128), f32),
    )(zd_t, zm_t, d_sim, m_sim, att_w1, att_b1, aw2,
      dfc_w, dfc_b, mfc_w, mfc_b, hfc_w, hfc_b, bil_w)

    sflat = table.reshape(TABLE_LEN)

    mesh = plsc.VectorSubcoreMesh(core_axis_name="c", subcore_axis_name="s",
                                  num_cores=NUM_CORES,
                                  num_subcores=NUM_SUBCORES)
    scores = pl.kernel(
        _sc_gather,
        out_type=jax.ShapeDtypeStruct((N_PAIRS,), f32),
        mesh=mesh,
        scratch_types=[
            pltpu.VMEM((CHUNK,), jnp.int32),
            pltpu.VMEM((CHUNK,), jnp.int32),
            pltpu.VMEM((CHUNK,), jnp.int32),
            pltpu.VMEM((CHUNK,), f32),
            pltpu.VMEM_SHARED((TABLE_LEN,), f32),
            pltpu.SemaphoreType.DMA,
            pltpu.SemaphoreType.DMA,
            pltpu.SemaphoreType.DMA,
            pltpu.SemaphoreType.DMA,
        ],
    )(sflat, diseases.astype(jnp.int32), mirnas.astype(jnp.int32))

    return scores.reshape(N_PAIRS, 1)


# speedup vs baseline: 1.0060x; 1.0060x over previous
import jax
import jax.numpy as jnp
from jax import lax
from jax.experimental import pallas as pl
from jax.experimental.pallas import tpu as pltpu
from jax.experimental.pallas import tpu_sc as plsc

NUM_D = 383
NUM_M = 495
NUM_N = NUM_D + NUM_M
DIM = 128
HIDDEN = 512
N_PAIRS = 262144

ROW_PAD = 880
COL_PAD = 896
RB = ROW_PAD // 8
CB = COL_PAD // 128
N_TILES = RB * CB
TABLE_LEN = N_TILES * 1024

NUM_CORES = 2
NUM_SUBCORES = 16
NUM_WORKERS = NUM_CORES * NUM_SUBCORES
CHUNK = N_PAIRS // NUM_WORKERS
LANES = 16

N_SUB = 2
SUB = CHUNK // N_SUB


def _elu(x):
    return jnp.where(x > 0, x, jnp.exp(x) - 1.0)


def _tc_score_table(zd_ref, zm_ref, dsim_ref, msim_ref,
                    aw1_ref, ab1_ref, aw2_ref,
                    dfc_ref, db_ref, mfc_ref, mb_ref,
                    hw_ref, hb_ref, bil_ref, out_ref):
    bf16 = jnp.bfloat16
    aw1 = aw1_ref[...].astype(bf16)
    ab1 = ab1_ref[...]
    aw2 = aw2_ref[...][None, :]

    def attn(z_ref, n):
        betas = []
        for p in range(5):
            zp = z_ref[p]
            w = jnp.tanh(jnp.dot(zp.astype(bf16), aw1,
                                 preferred_element_type=jnp.float32) + ab1)
            s = jnp.sum(w * aw2, axis=1, keepdims=True)
            betas.append(jax.nn.sigmoid(jnp.sum(s) / n))
        h = betas[0] * z_ref[0]
        for p in range(1, 5):
            h = h + betas[p] * z_ref[p]
        return h

    h1 = attn(zd_ref, NUM_D)
    h2 = attn(zm_ref, NUM_M)

    h_d = _elu(jnp.dot(h1, dfc_ref[:DIM], preferred_element_type=jnp.float32)
               + jnp.dot(dsim_ref[...], dfc_ref[DIM:],
                         preferred_element_type=jnp.float32)
               + db_ref[...])
    h_m = _elu(jnp.dot(h2, mfc_ref[:DIM], preferred_element_type=jnp.float32)
               + jnp.dot(msim_ref[...], mfc_ref[DIM:],
                         preferred_element_type=jnp.float32)
               + mb_ref[...])

    pad2 = jnp.zeros((ROW_PAD - NUM_N, DIM), jnp.float32)
    h = jnp.concatenate([h_d, h_m, pad2], axis=0)
    h = _elu(jnp.dot(h, hw_ref[...], preferred_element_type=jnp.float32)
             + hb_ref[...])
    g = jnp.dot(h, bil_ref[...], preferred_element_type=jnp.float32)
    scores = lax.dot_general(g, h, (((1,), (1,)), ((), ())),
                             preferred_element_type=jnp.float32)
    scores = jax.nn.sigmoid(scores)
    scores = jnp.concatenate(
        [scores, jnp.zeros((ROW_PAD, COL_PAD - ROW_PAD), jnp.float32)],
        axis=1)
    for rb in range(RB):
        for cb in range(CB):
            out_ref[rb * CB + cb] = scores[8 * rb:8 * rb + 8,
                                           128 * cb:128 * cb + 128]


STAGE = TABLE_LEN // NUM_SUBCORES


def _sc_gather(sflat_hbm, d_hbm, m_hbm, out_hbm, d_v, m_v, idx_v, val_v, stab,
               ld_sem, tab_sem, g_sem, st_sem):
    sid = lax.axis_index("s")
    wid = sid * NUM_CORES + lax.axis_index("c")
    base = wid * CHUNK
    toff = pl.multiple_of(sid * STAGE, 8)
    tstage = pltpu.async_copy(sflat_hbm.at[pl.ds(toff, STAGE)],
                              stab.at[pl.ds(toff, STAGE)], tab_sem)
    ld_d = pltpu.async_copy(d_hbm.at[pl.ds(base, CHUNK)], d_v, ld_sem)
    ld_m = pltpu.async_copy(m_hbm.at[pl.ds(base, CHUNK)], m_v, ld_sem)
    ld_d.wait()
    ld_m.wait()

    vec_per_iter = 8
    n_iter = SUB // (LANES * vec_per_iter)

    gathers = []
    stores = []
    for k in range(N_SUB):
        kbase = k * SUB

        def body(i, carry, kbase=kbase):
            for j in range(vec_per_iter):
                off = pl.multiple_of(
                    kbase + i * (LANES * vec_per_iter) + j * LANES, LANES)
                r = d_v[pl.ds(off, LANES)]
                c = m_v[pl.ds(off, LANES)]
                tile = (r >> 3) * CB + (c >> 7)
                idx_v[pl.ds(off, LANES)] = ((tile << 10) + ((r & 7) << 7)
                                            + (c & 127))
            return carry

        lax.fori_loop(0, n_iter, body, 0)
        if k == 0:
            tstage.wait()
            plsc.subcore_barrier()
        gathers.append(pltpu.async_copy(
            stab.at[idx_v.at[pl.ds(kbase, SUB)]],
            val_v.at[pl.ds(kbase, SUB)], g_sem))
    for k in range(N_SUB):
        gathers[k].wait()
        stores.append(pltpu.async_copy(
            val_v.at[pl.ds(k * SUB, SUB)],
            out_hbm.at[pl.ds(base + k * SUB, SUB)], st_sem))
    for s in stores:
        s.wait()


def kernel(z_d, z_m, d_sim, m_sim, diseases, mirnas, att_w1, att_b1, att_w2,
           mfc_w, mfc_b, dfc_w, dfc_b, hfc_w, hfc_b, bil_w):
    f32 = jnp.float32

    zd_t = jnp.transpose(z_d, (1, 0, 2))
    zm_t = jnp.transpose(z_m, (1, 0, 2))
    aw2 = att_w2.reshape(HIDDEN)

    table = pl.pallas_call(
        _tc_score_table,
        out_shape=jax.ShapeDtypeStruct((N_TILES, 8, 128), f32),
    )(zd_t, zm_t, d_sim, m_sim, att_w1, att_b1, aw2,
      dfc_w, dfc_b, mfc_w, mfc_b, hfc_w, hfc_b, bil_w)

    sflat = table.reshape(TABLE_LEN)

    mesh = plsc.VectorSubcoreMesh(core_axis_name="c", subcore_axis_name="s",
                                  num_cores=NUM_CORES,
                                  num_subcores=NUM_SUBCORES)
    scores = pl.kernel(
        _sc_gather,
        out_type=jax.ShapeDtypeStruct((N_PAIRS,), f32),
        mesh=mesh,
        scratch_types=[
            pltpu.VMEM((CHUNK,), jnp.int32),
            pltpu.VMEM((CHUNK,), jnp.int32),
            pltpu.VMEM((CHUNK,), jnp.int32),
            pltpu.VMEM((CHUNK,), f32),
            pltpu.VMEM_SHARED((TABLE_LEN,), f32),
            pltpu.SemaphoreType.DMA,
            pltpu.SemaphoreType.DMA,
            pltpu.SemaphoreType.DMA,
            pltpu.SemaphoreType.DMA,
        ],
    )(sflat, diseases.astype(jnp.int32), mirnas.astype(jnp.int32))

    return scores.reshape(N_PAIRS, 1)
